# baseline (speedup 1.0000x reference)
import jax
import jax.numpy as jnp
from jax import lax
from jax.experimental import pallas as pl
from jax.experimental.pallas import tpu as pltpu

_T = 1024


def _kmeans_body(iters_ref, xa_ref, cb0t_ref, out_ref, cbt_ref):
    n = xa_ref.shape[0]
    d = out_ref.shape[1]
    kk = cb0t_ref.shape[1]
    nt = n // _T
    cbt_ref[:] = cb0t_ref[:]

    iota1 = lax.broadcasted_iota(jnp.int32, (_T, kk), 1)

    def outer(_, carry):
        cbt = cbt_ref[:]
        c2 = jnp.sum(cbt * cbt, axis=0, keepdims=True)
        cbt16 = cbt.astype(jnp.bfloat16)

        def argmin_onehot(j):
            xa = xa_ref[pl.ds(j * _T, _T), :]
            x16 = xa[:, :d].astype(jnp.bfloat16)
            g = lax.dot_general(x16, cbt16, (((1,), (0,)), ((), ())),
                                preferred_element_type=jnp.float32)
            dt = c2 - (g + g)
            m = jnp.min(dt, axis=1, keepdims=True)
            idx = jnp.min(jnp.where(dt == m, iota1, kk), axis=1, keepdims=True)
            return (iota1 == idx).astype(jnp.float32)

        def scatter_dot(su, j, oh):
            xa = xa_ref[pl.ds(j * _T, _T), :]
            return su + lax.dot_general(xa, oh, (((0,), (0,)), ((), ())),
                                        preferred_element_type=jnp.float32,
                                        precision=lax.Precision.HIGHEST)

        def tile(j, carry):
            su, oh_prev = carry
            su = scatter_dot(su, jnp.maximum(j - 1, 0), oh_prev)
            oh = argmin_onehot(j)
            return su, oh

        su0 = jnp.zeros((d + 1, kk), jnp.float32)
        su, oh_last = lax.fori_loop(0, nt, tile,
                                    (su0, jnp.zeros((_T, kk), jnp.float32)))
        su = scatter_dot(su, nt - 1, oh_last)
        sums, counts = su[:d, :], su[d:, :]
        mean = sums / jnp.maximum(counts, 1.0)
        cbt_ref[:] = jnp.where(counts > 0.0, mean, cbt)
        return carry

    lax.fori_loop(0, iters_ref[0], outer, 0)
    out_ref[:] = cbt_ref[:].T


def kernel(X, codebook, iters):
    n, d = X.shape
    kk = codebook.shape[0]
    idx = jax.random.permutation(jax.random.key(42), n)[:kk]
    cb0t = X[idx].T
    xa = jnp.concatenate([X, jnp.ones((n, 1), X.dtype)], axis=1)
    it = jnp.asarray(iters, jnp.int32).reshape(1)
    return pl.pallas_call(
        _kmeans_body,
        out_shape=jax.ShapeDtypeStruct((kk, d), X.dtype),
        in_specs=[
            pl.BlockSpec(memory_space=pltpu.SMEM),
            pl.BlockSpec(memory_space=pltpu.VMEM),
            pl.BlockSpec(memory_space=pltpu.VMEM),
        ],
        out_specs=pl.BlockSpec(memory_space=pltpu.VMEM),
        scratch_shapes=[pltpu.VMEM((d, kk), jnp.float32)],
    )(it, xa, cb0t)


# speedup vs baseline: 1.1710x; 1.1710x over previous
import jax
import jax.numpy as jnp
from jax import lax
from jax.experimental import pallas as pl
from jax.experimental.pallas import tpu as pltpu

_T = 1024


def _kmeans_body(iters_ref, xa_ref, cb0t_ref, out_ref, cbt_ref):
    n = xa_ref.shape[0]
    d = out_ref.shape[1]
    kk = cb0t_ref.shape[1]
    nt = n // _T
    cbt_ref[:] = cb0t_ref[:]

    iota1 = lax.broadcasted_iota(jnp.int32, (_T, kk), 1)

    def outer(_, carry):
        cbt = cbt_ref[:]
        c2 = jnp.sum(cbt * cbt, axis=0, keepdims=True)
        cbt16x2 = cbt.astype(jnp.bfloat16) * jnp.bfloat16(2.0)

        def tile(j, su):
            xa = xa_ref[pl.ds(j * _T, _T), :]
            x16 = xa[:, :d].astype(jnp.bfloat16)
            g2 = lax.dot_general(x16, cbt16x2, (((1,), (0,)), ((), ())),
                                 preferred_element_type=jnp.float32)
            dt = c2 - g2
            m = jnp.min(dt, axis=1, keepdims=True)
            idx = jnp.min(jnp.where(dt == m, iota1, kk), axis=1, keepdims=True)
            oh = (iota1 == idx).astype(jnp.float32)
            return su + lax.dot_general(xa, oh, (((0,), (0,)), ((), ())),
                                        preferred_element_type=jnp.float32,
                                        precision=lax.Precision.HIGHEST)

        su = lax.fori_loop(0, nt, tile, jnp.zeros((d + 1, kk), jnp.float32))
        sums, counts = su[:d, :], su[d:, :]
        mean = sums / jnp.maximum(counts, 1.0)
        cbt_ref[:] = jnp.where(counts > 0.0, mean, cbt)
        return carry

    lax.fori_loop(0, iters_ref[0], outer, 0)
    out_ref[:] = cbt_ref[:].T


def kernel(X, codebook, iters):
    n, d = X.shape
    kk = codebook.shape[0]
    idx = jax.random.permutation(jax.random.key(42), n)[:kk]
    cb0t = X[idx].T
    xa = jnp.concatenate([X, jnp.ones((n, 1), X.dtype)], axis=1)
    it = jnp.asarray(iters, jnp.int32).reshape(1)
    return pl.pallas_call(
        _kmeans_body,
        out_shape=jax.ShapeDtypeStruct((kk, d), X.dtype),
        in_specs=[
            pl.BlockSpec(memory_space=pltpu.SMEM),
            pl.BlockSpec(memory_space=pltpu.VMEM),
            pl.BlockSpec(memory_space=pltpu.VMEM),
        ],
        out_specs=pl.BlockSpec(memory_space=pltpu.VMEM),
        scratch_shapes=[pltpu.VMEM((d, kk), jnp.float32)],
    )(it, xa, cb0t)


# speedup vs baseline: 1.2238x; 1.0451x over previous
import jax
import jax.numpy as jnp
import numpy as np
from jax import lax
from jax.experimental import pallas as pl
from jax.experimental.pallas import tpu as pltpu

_T = 1024

_IDX_CACHE = {}


def _init_indices(n, kk):
    if (n, kk) not in _IDX_CACHE:
        with jax.ensure_compile_time_eval():
            perm = jax.random.permutation(jax.random.key(42), n)
        _IDX_CACHE[(n, kk)] = np.asarray(perm)[:kk]
    return _IDX_CACHE[(n, kk)]


def _kmeans_body(iters_ref, xa_ref, cb0t_ref, out_ref, cbt_ref):
    n = xa_ref.shape[0]
    d = out_ref.shape[1]
    kk = cb0t_ref.shape[1]
    nt = n // _T
    cbt_ref[:] = cb0t_ref[:]

    iota1 = lax.broadcasted_iota(jnp.int32, (_T, kk), 1)

    def outer(_, carry):
        cbt = cbt_ref[:]
        c2 = jnp.sum(cbt * cbt, axis=0, keepdims=True)
        cbt16x2 = cbt.astype(jnp.bfloat16) * jnp.bfloat16(2.0)

        def tile(j, su):
            xa = xa_ref[pl.ds(j * _T, _T), :]
            x16 = xa[:, :d].astype(jnp.bfloat16)
            g2 = lax.dot_general(x16, cbt16x2, (((1,), (0,)), ((), ())),
                                 preferred_element_type=jnp.float32)
            dt = c2 - g2
            m = jnp.min(dt, axis=1, keepdims=True)
            idx = jnp.min(jnp.where(dt == m, iota1, kk), axis=1, keepdims=True)
            oh = (iota1 == idx).astype(jnp.float32)
            return su + lax.dot_general(xa, oh, (((0,), (0,)), ((), ())),
                                        preferred_element_type=jnp.float32,
                                        precision=lax.Precision.HIGHEST)

        su = lax.fori_loop(0, nt, tile, jnp.zeros((d + 1, kk), jnp.float32))
        sums, counts = su[:d, :], su[d:, :]
        mean = sums / jnp.maximum(counts, 1.0)
        cbt_ref[:] = jnp.where(counts > 0.0, mean, cbt)
        return carry

    lax.fori_loop(0, iters_ref[0], outer, 0)
    out_ref[:] = cbt_ref[:].T


def kernel(X, codebook, iters):
    n, d = X.shape
    kk = codebook.shape[0]
    idx = jnp.asarray(_init_indices(n, kk))
    cb0t = X[idx].T
    xa = jnp.concatenate([X, jnp.ones((n, 1), X.dtype)], axis=1)
    it = jnp.asarray(iters, jnp.int32).reshape(1)
    return pl.pallas_call(
        _kmeans_body,
        out_shape=jax.ShapeDtypeStruct((kk, d), X.dtype),
        in_specs=[
            pl.BlockSpec(memory_space=pltpu.SMEM),
            pl.BlockSpec(memory_space=pltpu.VMEM),
            pl.BlockSpec(memory_space=pltpu.VMEM),
        ],
        out_specs=pl.BlockSpec(memory_space=pltpu.VMEM),
        scratch_shapes=[pltpu.VMEM((d, kk), jnp.float32)],
    )(it, xa, cb0t)
